# baseline (speedup 1.0000x reference)
import jax
import jax.numpy as jnp
from jax.experimental import pallas as pl
from jax.experimental.pallas import tpu as pltpu


def _se_kernel(x_ref, w1_ref, b1_ref, alpha_ref, w2_ref, b2_ref, o_ref):
    x = x_ref[0].astype(jnp.float32)
    hw = x.shape[1]

    scale_col = jnp.full((hw, 1), 1.0 / hw, dtype=jnp.float32)
    pooled = jnp.dot(x, scale_col,
                     preferred_element_type=jnp.float32)

    h = jnp.dot(w1_ref[...], pooled,
                preferred_element_type=jnp.float32) + b1_ref[...]
    h = jnp.where(h >= 0, h, alpha_ref[...] * h)
    y = jnp.dot(w2_ref[...], h,
                preferred_element_type=jnp.float32) + b2_ref[...]
    gate = jax.nn.sigmoid(y)

    o_ref[0] = (x * gate).astype(o_ref.dtype)


def kernel(x_nchw, w1, b1, alpha, w2, b2):
    N, C, H, W = x_nchw.shape
    HW = H * W
    Cr = w1.shape[0]

    x3 = x_nchw.reshape(N, C, HW)
    itemsize = jnp.dtype(x3.dtype).itemsize

    cost = pl.CostEstimate(
        flops=int(2 * N * C * HW + 4 * N * C * Cr),
        transcendentals=int(N * C),
        bytes_accessed=int(2 * N * C * HW * itemsize),
    )

    out3 = pl.pallas_call(
        _se_kernel,
        out_shape=jax.ShapeDtypeStruct((N, C, HW), x3.dtype),
        grid=(N,),
        in_specs=[
            pl.BlockSpec((1, C, HW), lambda i: (i, 0, 0)),
            pl.BlockSpec((Cr, C), lambda i: (0, 0)),
            pl.BlockSpec((Cr, 1), lambda i: (0, 0)),
            pl.BlockSpec((Cr, 1), lambda i: (0, 0)),
            pl.BlockSpec((C, Cr), lambda i: (0, 0)),
            pl.BlockSpec((C, 1), lambda i: (0, 0)),
        ],
        out_specs=pl.BlockSpec((1, C, HW), lambda i: (i, 0, 0)),
        compiler_params=pltpu.CompilerParams(
            dimension_semantics=("parallel",),
            vmem_limit_bytes=64 * 1024 * 1024,
        ),
        cost_estimate=cost,
    )(x3, w1, b1.reshape(Cr, 1), alpha.reshape(Cr, 1), w2, b2.reshape(C, 1))

    return out3.reshape(N, C, H, W)


# speedup vs baseline: 1.0891x; 1.0891x over previous
import functools

import jax
import jax.numpy as jnp
from jax.experimental import pallas as pl
from jax.experimental.pallas import tpu as pltpu


def _ring_kernel(x_hbm, w1_ref, b1_ref, alpha_ref, w2_ref, b2_ref, o_hbm,
                 x_buf, o_buf, in_sems, out_sems,
                 *, n_chunks_per_core: int, nb: int, depth: int):
    core = pl.program_id(0)
    base = core * n_chunks_per_core

    def start_in(slot, chunk):
        pltpu.make_async_copy(
            x_hbm.at[pl.ds((base + chunk) * nb, nb)],
            x_buf.at[slot],
            in_sems.at[slot],
        ).start()

    def wait_in(slot):
        pltpu.make_async_copy(
            x_hbm.at[pl.ds(0, nb)], x_buf.at[slot], in_sems.at[slot]
        ).wait()

    def start_out(slot, chunk):
        pltpu.make_async_copy(
            o_buf.at[slot],
            o_hbm.at[pl.ds((base + chunk) * nb, nb)],
            out_sems.at[slot],
        ).start()

    def wait_out(slot):
        pltpu.make_async_copy(
            o_buf.at[slot], o_hbm.at[pl.ds(0, nb)], out_sems.at[slot]
        ).wait()

    for s in range(depth):
        start_in(s, s)

    hw = x_buf.shape[-1]
    scale_col = jnp.full((hw, 1), 1.0 / hw, dtype=jnp.float32)

    def body(i, _):
        slot = jax.lax.rem(i, depth)
        wait_in(slot)
        x = x_buf[slot].astype(jnp.float32)
        cols = [jnp.dot(x[n], scale_col, preferred_element_type=jnp.float32)
                for n in range(nb)]
        pooled = cols[0] if nb == 1 else jnp.concatenate(cols, axis=1)
        h = jnp.dot(w1_ref[...], pooled,
                    preferred_element_type=jnp.float32) + b1_ref[...]
        h = jnp.where(h >= 0, h, alpha_ref[...] * h)
        y = jnp.dot(w2_ref[...], h,
                    preferred_element_type=jnp.float32) + b2_ref[...]
        gate = jax.nn.sigmoid(y)

        @pl.when(i >= depth)
        def _():
            wait_out(slot)

        for n in range(nb):
            o_buf[slot, n] = (x[n] * gate[:, n:n + 1]).astype(o_buf.dtype)
        start_out(slot, i)

        @pl.when(i + depth < n_chunks_per_core)
        def _():
            start_in(slot, i + depth)
        return ()

    jax.lax.fori_loop(0, n_chunks_per_core, body, (), unroll=False)

    for k in range(depth):
        c = n_chunks_per_core - depth + k
        wait_out(jax.lax.rem(c, depth))


def kernel(x_nchw, w1, b1, alpha, w2, b2):
    N, C, H, W = x_nchw.shape
    HW = H * W
    Cr = w1.shape[0]
    x3 = x_nchw.reshape(N, C, HW)
    itemsize = jnp.dtype(x3.dtype).itemsize

    n_cores = 2
    nb = 2
    depth = 3
    n_chunks_per_core = N // (nb * n_cores)

    kern = functools.partial(
        _ring_kernel, n_chunks_per_core=n_chunks_per_core, nb=nb, depth=depth)

    cost = pl.CostEstimate(
        flops=int(2 * N * C * HW + 4 * N * C * Cr),
        transcendentals=int(N * C),
        bytes_accessed=int(2 * N * C * HW * itemsize),
    )

    out3 = pl.pallas_call(
        kern,
        out_shape=jax.ShapeDtypeStruct((N, C, HW), x3.dtype),
        grid=(n_cores,),
        in_specs=[
            pl.BlockSpec(memory_space=pl.ANY),
            pl.BlockSpec((Cr, C), lambda i: (0, 0)),
            pl.BlockSpec((Cr, 1), lambda i: (0, 0)),
            pl.BlockSpec((Cr, 1), lambda i: (0, 0)),
            pl.BlockSpec((C, Cr), lambda i: (0, 0)),
            pl.BlockSpec((C, 1), lambda i: (0, 0)),
        ],
        out_specs=pl.BlockSpec(memory_space=pl.ANY),
        scratch_shapes=[
            pltpu.VMEM((depth, nb, C, HW), x3.dtype),
            pltpu.VMEM((depth, nb, C, HW), x3.dtype),
            pltpu.SemaphoreType.DMA((depth,)),
            pltpu.SemaphoreType.DMA((depth,)),
        ],
        compiler_params=pltpu.CompilerParams(
            dimension_semantics=("parallel",),
            vmem_limit_bytes=64 * 1024 * 1024,
        ),
        cost_estimate=cost,
    )(x3, w1, b1.reshape(Cr, 1), alpha.reshape(Cr, 1), w2, b2.reshape(C, 1))

    return out3.reshape(N, C, H, W)
